# Initial kernel scaffold; baseline (speedup 1.0000x reference)
import jax
import jax.numpy as jnp
from jax.experimental import pallas as pl


def kernel(f0, f1, f2, f3, f4, f5, f6, f7, f8, f9, f10, f11, f12, f13, f14, f15, f16, f17, f18, f19, f20, f21, f22, f23, f24, f25, table_0, table_1, table_2, table_3, table_4, table_5, table_6, table_7, table_8, table_9, table_10, table_11, table_12, table_13, table_14, table_15, table_16, table_17, table_18, table_19, table_20, table_21, table_22, table_23, table_24, table_25):
    raise NotImplementedError("write your pallas kernel here")


# speedup vs baseline: 2.5240x; 2.5240x over previous
import functools

import jax
import jax.numpy as jnp
from jax import lax
from jax.experimental import pallas as pl
from jax.experimental.pallas import tpu as pltpu
from jax.experimental.pallas import tpu_sc as plsc

NUM_FIELDS = 26
EMBED = 16
VOCAB = 100000


@functools.lru_cache(maxsize=None)
def _build(N: int):
    info = plsc.get_sparse_core_info()
    NC, NS = info.num_cores, info.num_subcores
    NW = NC * NS
    assert N % (8 * NW) == 0
    n_per_w = N // NW

    mesh = plsc.VectorSubcoreMesh(core_axis_name="c", subcore_axis_name="s")

    @functools.partial(
        pl.kernel,
        mesh=mesh,
        compiler_params=pltpu.CompilerParams(use_tc_tiling_on_sc=False),
        out_type=jax.ShapeDtypeStruct((N, NUM_FIELDS, EMBED), jnp.float32),
        scratch_types=[
            pltpu.VMEM((n_per_w,), jnp.int32),
            pltpu.VMEM((n_per_w, EMBED), jnp.float32),
            pltpu.SemaphoreType.DMA,
        ],
    )
    def k(*refs):
        idx_hbm = refs[:NUM_FIELDS]
        tables = refs[NUM_FIELDS:2 * NUM_FIELDS]
        out = refs[2 * NUM_FIELDS]
        idx_v, rows_v, sem = refs[2 * NUM_FIELDS + 1:]

        wid = lax.axis_index("s") * NC + lax.axis_index("c")
        base = wid * n_per_w
        for i in range(NUM_FIELDS):
            pltpu.sync_copy(idx_hbm[i].at[pl.ds(base, n_per_w)], idx_v)
            pltpu.async_copy(tables[i].at[idx_v], rows_v, sem).wait()
            pltpu.sync_copy(rows_v, out.at[pl.ds(base, n_per_w), i])

    return k


def kernel(f0, f1, f2, f3, f4, f5, f6, f7, f8, f9, f10, f11, f12, f13, f14, f15, f16, f17, f18, f19, f20, f21, f22, f23, f24, f25, table_0, table_1, table_2, table_3, table_4, table_5, table_6, table_7, table_8, table_9, table_10, table_11, table_12, table_13, table_14, table_15, table_16, table_17, table_18, table_19, table_20, table_21, table_22, table_23, table_24, table_25):
    fs = [f0, f1, f2, f3, f4, f5, f6, f7, f8, f9, f10, f11, f12, f13, f14,
          f15, f16, f17, f18, f19, f20, f21, f22, f23, f24, f25]
    tables = [table_0, table_1, table_2, table_3, table_4, table_5, table_6,
              table_7, table_8, table_9, table_10, table_11, table_12,
              table_13, table_14, table_15, table_16, table_17, table_18,
              table_19, table_20, table_21, table_22, table_23, table_24,
              table_25]
    B, L = fs[0].shape
    N = B * L
    flat = [f.reshape(N) for f in fs]
    out = _build(N)(*flat, *tables)
    return out.reshape(B, L, NUM_FIELDS * EMBED)


# speedup vs baseline: 2.5720x; 1.0190x over previous
import functools

import jax
import jax.numpy as jnp
from jax import lax
from jax.experimental import pallas as pl
from jax.experimental.pallas import tpu as pltpu
from jax.experimental.pallas import tpu_sc as plsc

NUM_FIELDS = 26
EMBED = 16
VOCAB = 100000


@functools.lru_cache(maxsize=None)
def _build(N: int):
    info = plsc.get_sparse_core_info()
    NC, NS = info.num_cores, info.num_subcores
    NW = NC * NS
    assert N % (8 * NW) == 0
    n_per_w = N // NW

    mesh = plsc.VectorSubcoreMesh(core_axis_name="c", subcore_axis_name="s")

    @functools.partial(
        pl.kernel,
        mesh=mesh,
        compiler_params=pltpu.CompilerParams(use_tc_tiling_on_sc=False),
        out_type=jax.ShapeDtypeStruct((N, NUM_FIELDS * EMBED), jnp.float32),
        scratch_types=[
            pltpu.VMEM((2, n_per_w), jnp.int32),
            pltpu.VMEM((2, n_per_w, EMBED), jnp.float32),
            pltpu.SemaphoreType.DMA((2,)),
            pltpu.SemaphoreType.DMA((2,)),
            pltpu.SemaphoreType.DMA((2,)),
        ],
    )
    def k(idx_hbm, *refs):
        tables = refs[:NUM_FIELDS]
        out = refs[NUM_FIELDS]
        idx_v, rows_v, isem, gsem, wsem = refs[NUM_FIELDS + 1:]

        wid = lax.axis_index("s") * NC + lax.axis_index("c")
        base = wid * n_per_w

        def idx_start(i):
            p = i & 1
            return pltpu.async_copy(
                idx_hbm.at[pl.ds(i * N + base, n_per_w)], idx_v.at[p],
                isem.at[p])

        def gather_start(i):
            p = i & 1
            return pltpu.async_copy(
                tables[i].at[idx_v.at[p]], rows_v.at[p], gsem.at[p])

        def write_start(i):
            p = i & 1
            return pltpu.async_copy(
                rows_v.at[p],
                out.at[pl.ds(base, n_per_w), pl.ds(EMBED * i, EMBED)],
                wsem.at[p])

        idx_h = [idx_start(0), None]
        idx_h[0].wait()
        g_h = [gather_start(0), None]
        idx_h[1] = idx_start(1)
        w_h = [None, None]
        for i in range(NUM_FIELDS):
            p = i & 1
            q = 1 - p
            if i + 1 < NUM_FIELDS:
                if w_h[q] is not None:
                    w_h[q].wait()
                idx_h[q].wait()
                g_h[q] = gather_start(i + 1)
            g_h[p].wait()
            if i + 2 < NUM_FIELDS:
                idx_h[p] = idx_start(i + 2)
            w_h[p] = write_start(i)
        w_h[0].wait()
        w_h[1].wait()

    return k


def kernel(f0, f1, f2, f3, f4, f5, f6, f7, f8, f9, f10, f11, f12, f13, f14, f15, f16, f17, f18, f19, f20, f21, f22, f23, f24, f25, table_0, table_1, table_2, table_3, table_4, table_5, table_6, table_7, table_8, table_9, table_10, table_11, table_12, table_13, table_14, table_15, table_16, table_17, table_18, table_19, table_20, table_21, table_22, table_23, table_24, table_25):
    fs = [f0, f1, f2, f3, f4, f5, f6, f7, f8, f9, f10, f11, f12, f13, f14,
          f15, f16, f17, f18, f19, f20, f21, f22, f23, f24, f25]
    tables = [table_0, table_1, table_2, table_3, table_4, table_5, table_6,
              table_7, table_8, table_9, table_10, table_11, table_12,
              table_13, table_14, table_15, table_16, table_17, table_18,
              table_19, table_20, table_21, table_22, table_23, table_24,
              table_25]
    B, L = fs[0].shape
    N = B * L
    idx_flat = jnp.concatenate([f.reshape(N) for f in fs])
    out = _build(N)(idx_flat, *tables)
    return out.reshape(B, L, NUM_FIELDS * EMBED)
